# baseline (speedup 1.0000x reference)
import jax
import jax.numpy as jnp
from jax import lax
from jax.experimental import pallas as pl

_BB = 32


def _bcast_kernel(dummy_ref, pos_ref, out_ref):
    del dummy_ref
    row = pos_ref[...]
    out_ref[...] = jnp.broadcast_to(row[None], out_ref.shape)


def kernel(dummy, position_ids):
    b = dummy.shape[0]
    _, c, one, s = position_ids.shape
    w = (c * s) // 32
    table = lax.bitcast_convert_type(
        position_ids.reshape(16, w, 2), jnp.int32
    )
    idx2d = dummy.reshape(1, b)
    grid = b // _BB
    out = pl.pallas_call(
        _bcast_kernel,
        grid=(grid,),
        in_specs=[
            pl.BlockSpec((1, b), lambda i: (0, 0)),
            pl.BlockSpec((16, w), lambda i: (0, 0)),
        ],
        out_specs=pl.BlockSpec((_BB, 16, w), lambda i: (i, 0, 0)),
        out_shape=jax.ShapeDtypeStruct((b, 16, w), jnp.int32),
    )(idx2d, table)
    out16 = lax.bitcast_convert_type(out, position_ids.dtype)
    return out16.reshape(b, c, one, s)


# speedup vs baseline: 2.9197x; 2.9197x over previous
import jax
import jax.numpy as jnp
from jax import lax
from jax.experimental import pallas as pl

_BB = 32
_R = 48


def _bcast_kernel(dummy_ref, pos_ref, out_ref):
    del dummy_ref
    row = pos_ref[...]
    out_ref[...] = jnp.broadcast_to(row[None], out_ref.shape)


def kernel(dummy, position_ids):
    b = dummy.shape[0]
    _, c, one, s = position_ids.shape
    w = (c * s) // _R
    table = lax.bitcast_convert_type(
        position_ids.reshape(_R, w), jnp.bfloat16
    )
    idx2d = dummy.reshape(1, b)
    grid = b // _BB
    out = pl.pallas_call(
        _bcast_kernel,
        grid=(grid,),
        in_specs=[
            pl.BlockSpec((1, b), lambda i: (0, 0)),
            pl.BlockSpec((_R, w), lambda i: (0, 0)),
        ],
        out_specs=pl.BlockSpec((_BB, _R, w), lambda i: (i, 0, 0)),
        out_shape=jax.ShapeDtypeStruct((b, _R, w), jnp.bfloat16),
    )(idx2d, table)
    out16 = lax.bitcast_convert_type(out, position_ids.dtype)
    return out16.reshape(b, c, one, s)


# speedup vs baseline: 7.5307x; 2.5793x over previous
import jax
import jax.numpy as jnp
from jax import lax
from jax.experimental import pallas as pl

_BB = 32


def _bcast_kernel(dummy_ref, pos_ref, out_ref):
    del dummy_ref
    c, bb, s = out_ref.shape
    for j in range(c):
        row = pos_ref[pl.ds(j, 1), :]
        out_ref[j] = jnp.broadcast_to(row, (bb, s))


def kernel(dummy, position_ids):
    b = dummy.shape[0]
    _, c, one, s = position_ids.shape
    table = lax.bitcast_convert_type(position_ids.reshape(c, s), jnp.bfloat16)
    idx2d = dummy.reshape(1, b)
    grid = b // _BB
    out = pl.pallas_call(
        _bcast_kernel,
        grid=(grid,),
        in_specs=[
            pl.BlockSpec((1, b), lambda i: (0, 0)),
            pl.BlockSpec((c, s), lambda i: (0, 0)),
        ],
        out_specs=pl.BlockSpec((c, _BB, s), lambda i: (0, i, 0)),
        out_shape=jax.ShapeDtypeStruct((c, b, s), jnp.bfloat16),
    )(idx2d, table)
    out16 = lax.bitcast_convert_type(out, position_ids.dtype)
    return jnp.transpose(out16, (1, 0, 2)).reshape(b, c, one, s)
